# baseline (speedup 1.0000x reference)
import functools

import jax
import jax.numpy as jnp
from jax import lax
from jax.experimental import pallas as pl
from jax.experimental.pallas import tpu as pltpu
from jax.experimental.pallas import tpu_sc as plsc

NC = 2
NS = 16
NW = NC * NS
CH = 128
NB = 4
L = 16


@functools.partial(jax.jit, static_argnames=("dim",))
def _gather_native(embr, idxT, dim):
    F, B = idxT.shape
    units = F * (B // CH)
    upt = units // NW
    jt = B // CH
    rpw = 128 // dim
    mesh = plsc.VectorSubcoreMesh(
        core_axis_name="c", subcore_axis_name="s", num_cores=NC, num_subcores=NS
    )

    @functools.partial(
        pl.kernel,
        out_type=jax.ShapeDtypeStruct((F, dim, B), jnp.float32),
        mesh=mesh,
        compiler_params=pltpu.CompilerParams(
            use_tc_tiling_on_sc=True, needs_layout_passes=False
        ),
        scratch_types=[
            pltpu.VMEM((NB, CH), jnp.int32),
            pltpu.VMEM((NB, CH), jnp.int32),
            pltpu.VMEM((NB, CH), jnp.int32),
            pltpu.VMEM((NB, CH, 128), jnp.float32),
            pltpu.VMEM((NB, dim, CH), jnp.float32),
        ]
        + [pltpu.SemaphoreType.DMA] * (3 * NB),
    )
    def gather_k(embr_hbm, idxT_hbm, out_hbm, idx_v, q_v, off_v, g_v, dst_v, *sems):
        isems = sems[:NB]
        gsems = sems[NB : 2 * NB]
        ssems = sems[2 * NB :]
        wid = lax.axis_index("s") * NC + lax.axis_index("c")
        u0 = wid * upt

        def unit_fj(u):
            gu = u0 + u
            return gu // jt, (gu % jt) * CH

        def idx_start(u, s):
            f, b = unit_fj(u)
            pltpu.async_copy(idxT_hbm.at[f, pl.ds(b, CH)], idx_v.at[s], isems[s])

        def idx_wait(u, s):
            f, b = unit_fj(u)
            pltpu.make_async_copy(
                idxT_hbm.at[f, pl.ds(b, CH)], idx_v.at[s], isems[s]
            ).wait()

        qshift = rpw.bit_length() - 1
        dshift = dim.bit_length() - 1

        def prep(s):
            for k in range(CH // L):
                v = idx_v[s, pl.ds(k * L, L)]
                q_v[s, pl.ds(k * L, L)] = lax.shift_right_logical(v, qshift)
                off_v[s, pl.ds(k * L, L)] = lax.shift_left(
                    lax.bitwise_and(v, rpw - 1), dshift
                )

        def gather_start(s):
            pltpu.async_copy(embr_hbm.at[q_v.at[s]], g_v.at[s], gsems[s])

        def gather_wait(s):
            pltpu.make_async_copy(embr_hbm.at[q_v.at[s]], g_v.at[s], gsems[s]).wait()

        def transpose(s):
            slot = jnp.full((L,), s, jnp.int32)

            def body(c, carry):
                for k in range(CH // L):
                    rows = lax.iota(jnp.int32, L) + (k * L)
                    off = off_v[s, pl.ds(k * L, L)]
                    vals = plsc.load_gather(g_v, [slot, rows, off + c])
                    dst_v[s, c, pl.ds(k * L, L)] = vals
                return carry

            lax.fori_loop(0, dim, body, 0, unroll=False)

        def store_start(u, s):
            f, b = unit_fj(u)
            pltpu.async_copy(
                dst_v.at[s], out_hbm.at[f, :, pl.ds(b, CH)], ssems[s]
            )

        def store_wait(u, s):
            f, b = unit_fj(u)
            pltpu.make_async_copy(
                dst_v.at[s], out_hbm.at[f, :, pl.ds(b, CH)], ssems[s]
            ).wait()

        for s in range(NB):
            idx_start(s, s)
        for uu in range(2):
            idx_wait(uu, uu)
            prep(uu)
            gather_start(uu)

        for su in range(NB):
            u = su
            gather_wait(su)
            transpose(su)
            store_start(u, su)
            if u + 2 <= upt - 1:
                s2 = (su + 2) % NB
                idx_wait(u + 2, s2)
                prep(s2)
                gather_start(s2)
            if u + NB <= upt - 1:
                idx_start(u + NB, su)

        groups = upt // NB

        def body(g, carry):
            for su in range(NB):
                u = g * NB + su
                gather_wait(su)
                store_wait(u - NB, su)
                transpose(su)
                store_start(u, su)
                s2 = (su + 2) % NB
                idx_wait(u + 2, s2)
                prep(s2)
                gather_start(s2)
                idx_start(u + NB, su)
            return carry

        lax.fori_loop(1, groups - 1, body, 0, unroll=False)

        for su in range(NB):
            u = (groups - 1) * NB + su
            gather_wait(su)
            store_wait(u - NB, su)
            transpose(su)
            store_start(u, su)
            if u + 2 <= upt - 1:
                s2 = (su + 2) % NB
                idx_wait(u + 2, s2)
                prep(s2)
                gather_start(s2)

        for su in range(NB):
            u = (groups - 1) * NB + su
            store_wait(u, su)

    return gather_k(embr, idxT)


def kernel(idx, emb):
    B, F = idx.shape
    V, D = emb.shape
    embr = emb.reshape(V * D // 128, 128)
    idxT = idx.T.astype(jnp.int32)
    outk = _gather_native(embr, idxT, D)
    return outk.transpose(2, 0, 1)


# speedup vs baseline: 1.3236x; 1.3236x over previous
import functools

import jax
import jax.numpy as jnp
from jax import lax
from jax.experimental import pallas as pl
from jax.experimental.pallas import tpu as pltpu
from jax.experimental.pallas import tpu_sc as plsc

NC = 2
NS = 16
NW = NC * NS
CH = 128
NB = 4
L = 16


def _sc_params():
    return pltpu.CompilerParams(use_tc_tiling_on_sc=True, needs_layout_passes=False)


def _mesh():
    return plsc.VectorSubcoreMesh(
        core_axis_name="c", subcore_axis_name="s", num_cores=NC, num_subcores=NS
    )


@jax.jit
def _relayout_table(embT, tailp):
    D, V = embT.shape
    nfull = V // 128
    trows = tailp.shape[0]
    R = V * D // 128
    base = nfull // NW
    rem = nfull - base * NW
    rpw = 128 // D

    @functools.partial(
        pl.kernel,
        out_type=jax.ShapeDtypeStruct((R, 128), jnp.float32),
        mesh=_mesh(),
        compiler_params=_sc_params(),
        scratch_types=[
            pltpu.VMEM((NB, D, 128), jnp.float32),
            pltpu.VMEM((NB, D, 128), jnp.float32),
            pltpu.SemaphoreType.DMA((NB,)),
            pltpu.SemaphoreType.DMA((NB,)),
        ],
    )
    def k1(embT_hbm, tailp_hbm, out_hbm, src_v, dst_v, lsem, ssem):
        wid = lax.axis_index("s") * NC + lax.axis_index("c")
        cnt = base + jnp.where(wid < rem, 1, 0)
        tc0 = base * wid + lax.min(wid, rem)
        riota = lax.iota(jnp.int32, L)

        def load_start(tc, s):
            pltpu.async_copy(
                embT_hbm.at[:, pl.ds(tc * 128, 128)],
                src_v.at[s],
                lsem.at[s],
            )

        def load_wait(tc, s):
            pltpu.make_async_copy(
                embT_hbm.at[:, pl.ds(tc * 128, 128)],
                src_v.at[s],
                lsem.at[s],
            ).wait()

        def transpose(s):
            for sr in range(D):
                vals = []
                for k in range(128 // L):
                    c0 = (k * L) % D
                    col = jnp.full((L,), rpw * sr + (k * L) // D, jnp.int32)
                    vals.append(plsc.load_gather(src_v.at[s], [c0 + riota, col]))
                for k in range(128 // L):
                    dst_v[s, sr, pl.ds(k * L, L)] = vals[k]

        def store_start(tc, s):
            pltpu.async_copy(
                dst_v.at[s], out_hbm.at[pl.ds(tc * D, D), :], ssem.at[s]
            )

        def store_drain(s):
            pltpu.make_async_copy(
                dst_v.at[s], out_hbm.at[pl.ds(0, D), :], ssem.at[s]
            ).wait()

        for s in range(NB):
            load_start(tc0 + s, s)

        gmax = (base + (1 if rem else 0) + NB - 1) // NB

        def body(g, carry):
            for su in range(NB):
                u = g * NB + su

                @pl.when(u < cnt)
                def _(u=u, su=su):
                    @pl.when(u >= NB)
                    def _():
                        store_drain(su)

                    load_wait(tc0 + u, su)
                    transpose(su)
                    store_start(tc0 + u, su)

                    @pl.when(u + NB <= cnt - 1)
                    def _():
                        load_start(tc0 + u + NB, su)

            return carry

        lax.fori_loop(0, gmax, body, 0, unroll=False)

        for s in range(NB):
            store_drain(s)

        if trows:
            @pl.when(wid == NW - 1)
            def _tail():
                pltpu.sync_copy(tailp_hbm, src_v.at[0, pl.ds(0, trows), :])
                pltpu.sync_copy(
                    src_v.at[0, pl.ds(0, trows), :],
                    out_hbm.at[pl.ds(nfull * D, trows), :],
                )

    return k1(embT, tailp)


@functools.partial(jax.jit, static_argnames=("dim",))
def _gather_native(embr, idxT, dim):
    F, B = idxT.shape
    units = F * (B // CH)
    upt = units // NW
    jt = B // CH
    rpw = 128 // dim
    qshift = rpw.bit_length() - 1
    dshift = dim.bit_length() - 1

    @functools.partial(
        pl.kernel,
        out_type=jax.ShapeDtypeStruct((F, dim, B), jnp.float32),
        mesh=_mesh(),
        compiler_params=_sc_params(),
        scratch_types=[
            pltpu.VMEM((NB, CH), jnp.int32),
            pltpu.VMEM((NB, CH), jnp.int32),
            pltpu.VMEM((NB, CH), jnp.int32),
            pltpu.VMEM((NB, CH, 128), jnp.float32),
            pltpu.VMEM((NB, dim, CH), jnp.float32),
            pltpu.SemaphoreType.DMA((NB,)),
            pltpu.SemaphoreType.DMA((NB,)),
            pltpu.SemaphoreType.DMA((NB,)),
        ],
    )
    def k2(embr_hbm, idxT_hbm, out_hbm, idx_v, q_v, off_v, g_v, dst_v, isem, gsem, ssem):
        wid = lax.axis_index("s") * NC + lax.axis_index("c")
        u0 = wid * upt
        riota = lax.iota(jnp.int32, L)

        def unit_fj(u):
            gu = u0 + u
            return gu // jt, (gu % jt) * CH

        def idx_start(u, s):
            f, b = unit_fj(u)
            pltpu.async_copy(idxT_hbm.at[f, pl.ds(b, CH)], idx_v.at[s], isem.at[s])

        def idx_wait(u, s):
            f, b = unit_fj(u)
            pltpu.make_async_copy(
                idxT_hbm.at[f, pl.ds(b, CH)], idx_v.at[s], isem.at[s]
            ).wait()

        def prep(s):
            for k in range(CH // L):
                v = idx_v[s, pl.ds(k * L, L)]
                q_v[s, pl.ds(k * L, L)] = lax.shift_right_logical(v, qshift)
                off_v[s, pl.ds(k * L, L)] = lax.shift_left(
                    lax.bitwise_and(v, rpw - 1), dshift
                )

        def gather_start(s):
            pltpu.async_copy(embr_hbm.at[q_v.at[s]], g_v.at[s], gsem.at[s])

        def gather_wait(s):
            pltpu.make_async_copy(embr_hbm.at[q_v.at[s]], g_v.at[s], gsem.at[s]).wait()

        def transpose(s):
            for k in range(CH // L):
                rows = riota + (k * L)
                off = off_v[s, pl.ds(k * L, L)]
                for cb in range(0, dim, 8):
                    vals = [
                        plsc.load_gather(g_v.at[s], [rows, off + (cb + i)])
                        for i in range(8)
                    ]
                    for i in range(8):
                        dst_v[s, cb + i, pl.ds(k * L, L)] = vals[i]

        def store_start(u, s):
            f, b = unit_fj(u)
            pltpu.async_copy(
                dst_v.at[s], out_hbm.at[f, :, pl.ds(b, CH)], ssem.at[s]
            )

        def store_drain(s):
            pltpu.make_async_copy(
                dst_v.at[s], out_hbm.at[0, :, pl.ds(0, CH)], ssem.at[s]
            ).wait()

        for s in range(NB):
            idx_start(s, s)
        for uu in range(2):
            idx_wait(uu, uu)
            prep(uu)
            gather_start(uu)

        def body(g, carry):
            for su in range(NB):
                u = g * NB + su
                s2 = (su + 2) % NB

                @pl.when(u >= NB)
                def _(su=su):
                    store_drain(su)

                gather_wait(su)
                transpose(su)
                store_start(u, su)

                @pl.when(u + 2 <= upt - 1)
                def _(u=u, s2=s2):
                    idx_wait(u + 2, s2)
                    prep(s2)
                    gather_start(s2)

                @pl.when(u + NB <= upt - 1)
                def _(u=u, su=su):
                    idx_start(u + NB, su)

            return carry

        lax.fori_loop(0, upt // NB, body, 0, unroll=False)

        for s in range(NB):
            store_drain(s)

    return k2(embr, idxT)


def kernel(idx, emb):
    B, F = idx.shape
    V, D = emb.shape
    cut = (V // 128) * 128
    tailp = emb[cut:, :].reshape(-1, 128)
    embr = _relayout_table(emb.T, tailp)
    idxT = idx.T.astype(jnp.int32)
    outk = _gather_native(embr, idxT, D)
    return outk.transpose(2, 0, 1)
